# Initial kernel scaffold; baseline (speedup 1.0000x reference)
import jax
import jax.numpy as jnp
from jax.experimental import pallas as pl


def kernel(captions, lengths, table, W, b):
    raise NotImplementedError("write your pallas kernel here")


# speedup vs baseline: 1.4463x; 1.4463x over previous
import functools

import jax
import jax.numpy as jnp
from jax import lax
from jax.experimental import pallas as pl
from jax.experimental.pallas import tpu as pltpu
from jax.experimental.pallas import tpu_sc as plsc

VOCAB = 100000
EMBED = 128
SEQ = 22
OUT = 4800
BATCH = 4096

NC = 2
NS = 16
NW = NC * NS
TOTAL_IDX = BATCH * SEQ
IDX_PER_W = TOTAL_IDX // NW
CHUNKS = IDX_PER_W // 128


def _gather_body(idx_hbm, table_hbm, out_hbm, idx_v, buf_a, buf_b, sem_a, sem_b):
    wid = lax.axis_index("s") * NC + lax.axis_index("c")
    base = wid * IDX_PER_W
    pltpu.sync_copy(idx_hbm.at[wid], idx_v)
    bufs = (buf_a, buf_b)
    sems = (sem_a, sem_b)
    pltpu.make_async_copy(table_hbm.at[idx_v.at[0]], bufs[0], sems[0]).start()
    for j in range(1, CHUNKS + 1):
        if j < CHUNKS:
            pltpu.make_async_copy(
                table_hbm.at[idx_v.at[j]], bufs[j % 2], sems[j % 2]
            ).start()
        prev = j - 1
        pltpu.make_async_copy(
            table_hbm.at[idx_v.at[prev]], bufs[prev % 2], sems[prev % 2]
        ).wait()
        pltpu.sync_copy(
            bufs[prev % 2], out_hbm.at[pl.ds(base + prev * 128, 128)]
        )


@functools.lru_cache(maxsize=None)
def _make_gather():
    return functools.partial(
        pl.kernel,
        mesh=plsc.VectorSubcoreMesh(core_axis_name="c", subcore_axis_name="s"),
        out_type=jax.ShapeDtypeStruct((TOTAL_IDX, EMBED), jnp.float32),
        scratch_types=[
            pltpu.VMEM((CHUNKS, 128), jnp.int32),
            pltpu.VMEM((128, EMBED), jnp.float32),
            pltpu.VMEM((128, EMBED), jnp.float32),
            pltpu.SemaphoreType.DMA,
            pltpu.SemaphoreType.DMA,
        ],
    )(_gather_body)


def _mm_body(a_ref, w_ref, b_ref, o_ref):
    acc = lax.dot_general(
        a_ref[...], w_ref[...],
        dimension_numbers=(((1,), (1,)), ((), ())),
        preferred_element_type=jnp.float32,
    )
    o_ref[...] = jnp.maximum(acc + b_ref[...], 0.0)


def _matmul(flat, W, b2, bm, bn):
    k = flat.shape[1]
    grid = (BATCH // bm, pl.cdiv(OUT, bn))
    return pl.pallas_call(
        _mm_body,
        grid=grid,
        in_specs=[
            pl.BlockSpec((bm, k), lambda i, j: (i, 0)),
            pl.BlockSpec((bn, k), lambda i, j: (j, 0)),
            pl.BlockSpec((1, bn), lambda i, j: (0, j)),
        ],
        out_specs=pl.BlockSpec((bm, bn), lambda i, j: (i, j)),
        out_shape=jax.ShapeDtypeStruct((BATCH, OUT), jnp.float32),
    )(flat, W, b2)


def kernel(captions, lengths, table, W, b):
    idx = captions.reshape(NW, CHUNKS, 128).astype(jnp.int32)
    rows = _make_gather()(idx, table)
    flat = rows.reshape(BATCH, SEQ * EMBED)
    out = _matmul(flat, W, b.reshape(1, OUT), bm=512, bn=512)
    return out.reshape(BATCH, 3, 40, 40)


# speedup vs baseline: 1.6890x; 1.1678x over previous
import functools

import jax
import jax.numpy as jnp
from jax import lax
from jax.experimental import pallas as pl
from jax.experimental.pallas import tpu as pltpu
from jax.experimental.pallas import tpu_sc as plsc

VOCAB = 100000
EMBED = 128
SEQ = 22
OUT = 4800
BATCH = 4096

NC = 2
NS = 16
NW = NC * NS
TOTAL_IDX = BATCH * SEQ
IDX_PER_W = TOTAL_IDX // NW
CHUNKS = IDX_PER_W // 128


def _gather_body(idx_hbm, table_hbm, out_hbm, idx_v, buf_a, buf_b, sem_a, sem_b):
    wid = lax.axis_index("s") * NC + lax.axis_index("c")
    base = wid * IDX_PER_W
    pltpu.sync_copy(idx_hbm.at[wid], idx_v)
    bufs = (buf_a, buf_b)
    sems = (sem_a, sem_b)
    pltpu.make_async_copy(table_hbm.at[idx_v.at[0]], bufs[0], sems[0]).start()
    for j in range(1, CHUNKS + 1):
        if j < CHUNKS:
            pltpu.make_async_copy(
                table_hbm.at[idx_v.at[j]], bufs[j % 2], sems[j % 2]
            ).start()
        prev = j - 1
        pltpu.make_async_copy(
            table_hbm.at[idx_v.at[prev]], bufs[prev % 2], sems[prev % 2]
        ).wait()
        pltpu.sync_copy(
            bufs[prev % 2], out_hbm.at[pl.ds(base + prev * 128, 128)]
        )


@functools.lru_cache(maxsize=None)
def _make_gather():
    return functools.partial(
        pl.kernel,
        mesh=plsc.VectorSubcoreMesh(core_axis_name="c", subcore_axis_name="s"),
        out_type=jax.ShapeDtypeStruct((TOTAL_IDX, EMBED), jnp.float32),
        scratch_types=[
            pltpu.VMEM((CHUNKS, 128), jnp.int32),
            pltpu.VMEM((128, EMBED), jnp.float32),
            pltpu.VMEM((128, EMBED), jnp.float32),
            pltpu.SemaphoreType.DMA,
            pltpu.SemaphoreType.DMA,
        ],
    )(_gather_body)


def _mm_body(a_ref, w_ref, b_ref, o_ref):
    acc = lax.dot_general(
        a_ref[...].astype(jnp.bfloat16), w_ref[...],
        dimension_numbers=(((1,), (1,)), ((), ())),
        preferred_element_type=jnp.float32,
    )
    o_ref[...] = jnp.maximum(acc + b_ref[...], 0.0)


def _matmul(flat, W, b2, bm, bn):
    k = flat.shape[1]
    grid = (BATCH // bm, pl.cdiv(OUT, bn))
    return pl.pallas_call(
        _mm_body,
        grid=grid,
        in_specs=[
            pl.BlockSpec((bm, k), lambda i, j: (i, 0)),
            pl.BlockSpec((bn, k), lambda i, j: (j, 0)),
            pl.BlockSpec((1, bn), lambda i, j: (0, j)),
        ],
        out_specs=pl.BlockSpec((bm, bn), lambda i, j: (i, j)),
        out_shape=jax.ShapeDtypeStruct((BATCH, OUT), jnp.float32),
    )(flat, W, b2)


def kernel(captions, lengths, table, W, b):
    idx = captions.reshape(NW, CHUNKS, 128).astype(jnp.int32)
    rows = _make_gather()(idx, table)
    flat = rows.reshape(BATCH, SEQ * EMBED)
    out = _matmul(flat, W.astype(jnp.bfloat16), b.reshape(1, OUT), bm=1024, bn=1024)
    return out.reshape(BATCH, 3, 40, 40)
